# baseline (speedup 1.0000x reference)
import jax
import jax.numpy as jnp
from jax.experimental import pallas as pl
from jax.experimental.pallas import tpu as pltpu

_K = 64
_ROWS_PER_BLOCK = 16


def _topk_mask_body(x_ref, o_ref, cut_ref):
    x = x_ref[...]
    i = jax.lax.bitcast_convert_type(x, jnp.int32)
    key = jnp.where(i >= 0, i, jnp.bitwise_xor(jnp.bitwise_not(i), jnp.int32(-(2**31))))
    nrows = x.shape[0]
    kmin = jnp.min(key, axis=1, keepdims=True)
    kmax = jnp.max(key, axis=1, keepdims=True)
    ones_b = jnp.ones((x.shape[1], 1), jnp.bfloat16)

    def cnt_ge(t):
        ind = jnp.where(key >= t, 1.0, 0.0).astype(jnp.bfloat16)
        return jax.lax.dot_general(
            ind, ones_b, (((1,), (0,)), ((), ())),
            preferred_element_type=jnp.float32)

    def cond(carry):
        lo, hi, found, thr = carry
        return jnp.any((found == 0) & ((hi - 1) > lo))

    def body(carry):
        lo, hi, found, thr = carry
        mid = (lo >> 1) + (hi >> 1) + (lo & hi & 1)
        cnt = cnt_ge(mid)
        hit = cnt == float(_K)
        ge = cnt >= float(_K)
        thr = jnp.where(hit & (found == 0), mid, thr)
        found = found | hit.astype(jnp.int32)
        lo = jnp.where(ge, mid, lo)
        hi = jnp.where(ge, hi, mid)
        return lo, hi, found, thr

    carry0 = (
        kmin,
        kmax + 1,
        jnp.zeros((nrows, 1), jnp.int32),
        kmin,
    )
    lo, hi, found, thr = jax.lax.while_loop(cond, body, carry0)
    thr = jnp.where(found == 1, thr, lo)

    n_ge = cnt_ge(thr)
    col = jax.lax.broadcasted_iota(jnp.int32, x.shape, 1)
    cut_ref[...] = jnp.full((nrows, 1), jnp.iinfo(jnp.int32).max, jnp.int32)

    @pl.when(jnp.any(n_ge > float(_K)))
    def _():
        extra = n_ge.astype(jnp.int32) - _K
        tcol = jnp.where(key == thr, col, -1)
        cut = jnp.full((nrows, 1), jnp.iinfo(jnp.int32).max, jnp.int32)
        for _ in range(4):
            hi_col = jnp.max(jnp.where(tcol < cut, tcol, -1), axis=1, keepdims=True)
            cut = jnp.where(extra > 0, hi_col, cut)
            extra = jnp.maximum(extra - 1, 0)
        cut_ref[...] = cut

    cut = cut_ref[...]
    keep = (key > thr) | ((key == thr) & (col < cut))
    o_ref[...] = jnp.where(keep, jnp.maximum(x, 0.0), 0.0)


def kernel(x):
    m, n = x.shape
    grid = (m // _ROWS_PER_BLOCK,)
    return pl.pallas_call(
        _topk_mask_body,
        grid=grid,
        in_specs=[pl.BlockSpec((_ROWS_PER_BLOCK, n), lambda r: (r, 0))],
        out_specs=pl.BlockSpec((_ROWS_PER_BLOCK, n), lambda r: (r, 0)),
        out_shape=jax.ShapeDtypeStruct((m, n), x.dtype),
        scratch_shapes=[pltpu.VMEM((_ROWS_PER_BLOCK, 1), jnp.int32)],
        compiler_params=pltpu.CompilerParams(
            dimension_semantics=("arbitrary",),
        ),
    )(x)
